# baseline (speedup 1.0000x reference)
import functools

import jax
import jax.numpy as jnp
from jax import lax
from jax.experimental import pallas as pl
from jax.experimental.pallas import tpu as pltpu
from jax.experimental.pallas import tpu_sc as plsc

N = 10000
E = 160000
D_IN = 256
D_H = 256
D_E = 16

NT = 16
ET = E // NT
ETP = 10240
CH2 = 128
NCH2 = ETP // CH2
NB = 3
NPAD = 10240
STRIP = NPAD // NT
DQ = 64
NR = 4
DBLK = NPAD // NR
DSUB = DBLK // NT

_f32 = jnp.float32
_i32 = jnp.int32


def _mm_body(x_ref, w_ref, att_ref, h_ref, a2_ref):
    xb = x_ref[...]
    hb = jnp.dot(xb, w_ref[...], preferred_element_type=_f32)
    for q in range(4):
        h_ref[q] = hb[:, q * DQ:(q + 1) * DQ]
    asb = jnp.sum(hb * att_ref[0][None, :], axis=1)
    adb = jnp.sum(hb * att_ref[1][None, :], axis=1)
    a2_ref[...] = jnp.stack([asb, adb], axis=1)


_MMBLK = 1000

_mm = pl.pallas_call(
    _mm_body,
    grid=(N // _MMBLK,),
    in_specs=[
        pl.BlockSpec((_MMBLK, D_IN), lambda i: (i, 0)),
        pl.BlockSpec((D_IN, D_H), lambda i: (0, 0)),
        pl.BlockSpec((2, D_H), lambda i: (0, 0)),
    ],
    out_specs=[
        pl.BlockSpec((4, _MMBLK, DQ), lambda i: (0, i, 0)),
        pl.BlockSpec((_MMBLK, 2), lambda i: (i, 0)),
    ],
    out_shape=[
        jax.ShapeDtypeStruct((4, N, DQ), _f32),
        jax.ShapeDtypeStruct((N, 2), _f32),
    ],
)


def _ae_body(ea_ref, we_ref, att_ref, out_ref):
    wv = jnp.dot(we_ref[...], att_ref[...], preferred_element_type=_f32)
    out_ref[...] = jnp.dot(ea_ref[...], wv, preferred_element_type=_f32)


_AEBLK = 8000

_ae = pl.pallas_call(
    _ae_body,
    grid=(E // _AEBLK,),
    in_specs=[
        pl.BlockSpec((_AEBLK, D_E), lambda i: (i, 0)),
        pl.BlockSpec((D_E, D_H), lambda i: (0, 0)),
        pl.BlockSpec((D_H, 1), lambda i: (0, 0)),
    ],
    out_specs=pl.BlockSpec((_AEBLK, 1), lambda i: (i, 0)),
    out_shape=jax.ShapeDtypeStruct((E, 1), _f32),
)


@functools.lru_cache(maxsize=1)
def _build_sc():
  mesh = plsc.VectorSubcoreMesh(core_axis_name="c", subcore_axis_name="s")

  @functools.partial(
    pl.kernel,
    mesh=mesh,
    compiler_params=pltpu.CompilerParams(
        needs_layout_passes=False, use_tc_tiling_on_sc=False),
    out_type=(
        jax.ShapeDtypeStruct((NPAD,), _f32),
        jax.ShapeDtypeStruct((4, NPAD, DQ), _f32),
    ),
    scratch_types=[
        pltpu.VMEM((N,), _f32),
        pltpu.VMEM((N,), _f32),
        pltpu.VMEM((NPAD,), _f32),
        pltpu.VMEM((DSUB,), _f32),
        pltpu.VMEM((DSUB,), _f32),
        pltpu.VMEM((NCH2, CH2), _i32),
        pltpu.VMEM((NCH2, CH2), _i32),
        pltpu.VMEM((NCH2, CH2), _f32),
        pltpu.VMEM((NB, CH2, DQ), _f32),
        pltpu.SemaphoreType.DMA((NB,)),
        pltpu.SemaphoreType.DMA((NB,)),
        pltpu.VMEM_SHARED((NT, DBLK), _f32),
        pltpu.VMEM_SHARED((NPAD, DQ), _f32),
    ],
  )
  def _sc(src2_h, dst2_h, ae2_h, as_h, ad_h, h2_h,
          den_o, out_o,
          asv, adv, denp, strip, tmp,
          srcm2, dstm2, aefm, rows4, semg, sems,
          stageb, acc):
    c = lax.axis_index("c")
    s = lax.axis_index("s")
    zeros16 = jnp.zeros((16,), _f32)
    iota16 = lax.iota(_i32, 16)

    pltpu.sync_copy(src2_h.at[s], srcm2)
    pltpu.sync_copy(dst2_h.at[s], dstm2)
    pltpu.sync_copy(ae2_h.at[s], aefm)
    pltpu.sync_copy(as_h, asv)
    pltpu.sync_copy(ad_h, adv)

    def _z(i, carry):
        denp[pl.ds(i * 16, 16)] = zeros16
        return carry
    lax.fori_loop(0, NPAD // 16, _z, 0)

    def _p1(i, carry):
        e = iota16 + i * 16
        er = e // CH2
        ec = e % CH2
        si = plsc.load_gather(srcm2, [er, ec])
        di = plsc.load_gather(dstm2, [er, ec])
        a = (plsc.load_gather(asv, [si])
             + plsc.load_gather(adv, [di])
             + plsc.load_gather(aefm, [er, ec]))
        a = jnp.maximum(a, a * 0.2)
        ex = jnp.exp(a)
        plsc.store_scatter(aefm, [er, ec], ex)
        plsc.addupdate_scatter(denp, [di], ex)
        return carry
    lax.fori_loop(0, ETP // 16, _p1, 0)

    @pl.when(c == 0)
    def _combine():
        for k in range(NR):
            pltpu.sync_copy(denp.at[pl.ds(k * DBLK, DBLK)], stageb.at[s])
            plsc.subcore_barrier()
            sb = s * DSUB
            pltpu.sync_copy(stageb.at[0, pl.ds(sb, DSUB)], strip)

            def _red(j, carry):
                pltpu.sync_copy(stageb.at[j, pl.ds(sb, DSUB)], tmp)

                def _addv(i, carry2):
                    sl = pl.ds(i * 16, 16)
                    strip[sl] = strip[sl] + tmp[sl]
                    return carry2
                lax.fori_loop(0, DSUB // 16, _addv, 0)
                return carry
            lax.fori_loop(1, NT, _red, 0)
            pltpu.sync_copy(strip, den_o.at[pl.ds(k * DBLK + sb, DSUB)])
            plsc.subcore_barrier()

    def _pass(p, carry):
        qq = c * 2 + p
        h_q = h2_h.at[qq]
        out_q = out_o.at[qq]

        def _gather(g, b):
            return pltpu.async_copy(
                h_q.at[srcm2.at[g]], rows4.at[b], semg.at[b])

        def _gather_wait(g, b):
            pltpu.make_async_copy(
                h_q.at[srcm2.at[g]], rows4.at[b], semg.at[b]).wait()

        def _scatter(g, b):
            return pltpu.async_copy(
                rows4.at[b], acc.at[dstm2.at[g]], sems.at[b], add=True)

        def _scatter_wait(g, b):
            pltpu.make_async_copy(
                rows4.at[b], acc.at[dstm2.at[g]], sems.at[b]).wait()

        def _scale(g, b):
            gv = jnp.full((16,), g, _i32)

            @plsc.parallel_loop(0, CH2, unroll=4)
            def _row(r):
                sp = plsc.load_gather(aefm, [gv, jnp.full((16,), r, _i32)])
                for k in range(DQ // 16):
                    sl = pl.ds(k * 16, 16)
                    rows4[b, r, sl] = rows4[b, r, sl] * sp

        def _zr(r, carry2):
            for k in range(DQ // 16):
                rows4[0, r, pl.ds(k * 16, 16)] = zeros16
            return carry2
        lax.fori_loop(0, CH2, _zr, 0)
        zb = s * STRIP
        for k in range(STRIP // CH2):
            pltpu.sync_copy(rows4.at[0], acc.at[pl.ds(zb + k * CH2, CH2)])
        plsc.subcore_barrier()

        def _chunk(g, b):
            _gather_wait(g, b)
            _scale(g, b)
            _scatter(g, b)
            bp = (b + 2) % NB

            @pl.when(g >= NB - 2)
            def _svc_wait():
                _scatter_wait(g - (NB - 2), bp)

            @pl.when(g + 2 < NCH2)
            def _svc_gather():
                _gather(g + 2, bp)

        _gather(0, 0)
        _gather(1, 1)

        def _trip(q, carry2):
            for b in range(NB):
                _chunk(q * NB + b, b)
            return carry2
        lax.fori_loop(0, NCH2 // NB, _trip, 0)

        for g in range(NCH2 - NCH2 % NB, NCH2):
            _chunk(g, g % NB)
        for g in range(NCH2 - (NB - 2), NCH2):
            _scatter_wait(g, g % NB)

        plsc.subcore_barrier()
        ob = s * STRIP
        pltpu.sync_copy(acc.at[pl.ds(ob, STRIP)],
                        out_q.at[pl.ds(ob, STRIP)])
        plsc.subcore_barrier()
        return carry
    lax.fori_loop(0, 2, _pass, 0)

  return _sc


def _bn_body(acc_ref, den_ref, g_ref, b_ref, out_ref):
    j = pl.program_id(0)
    a = jnp.concatenate([acc_ref[0], acc_ref[1]], axis=1)
    d = den_ref[...] + 1e-16
    o = a / d
    mu = jnp.mean(o, axis=0, keepdims=True)
    var = jnp.mean((o - mu) ** 2, axis=0, keepdims=True)
    g = jnp.where(j == 0, g_ref[0:1, :], g_ref[1:2, :])
    b = jnp.where(j == 0, b_ref[0:1, :], b_ref[1:2, :])
    out_ref[...] = jnp.maximum(
        (o - mu) * lax.rsqrt(var + 1e-5) * g + b, 0.0)


_bn = pl.pallas_call(
    _bn_body,
    grid=(2,),
    in_specs=[
        pl.BlockSpec((2, N, DQ), lambda j: (j, 0, 0)),
        pl.BlockSpec((N, 1), lambda j: (0, 0)),
        pl.BlockSpec((2, 2 * DQ), lambda j: (0, 0)),
        pl.BlockSpec((2, 2 * DQ), lambda j: (0, 0)),
    ],
    out_specs=pl.BlockSpec((N, 2 * DQ), lambda j: (0, j)),
    out_shape=jax.ShapeDtypeStruct((N, D_H), _f32),
)


def kernel(x, edge_index, edge_attr, batch, W, att_src, att_dst, W_e,
           att_edge, bias, gamma, beta):
    del bias
    pad = ETP - ET
    src2 = jnp.concatenate(
        [edge_index[0].reshape(NT, ET),
         jnp.zeros((NT, pad), _i32)], axis=1).reshape(NT, NCH2, CH2)
    dst2 = jnp.concatenate(
        [edge_index[1].reshape(NT, ET),
         jnp.full((NT, pad), N, _i32)], axis=1).reshape(NT, NCH2, CH2)
    h4, a2 = _mm(x, W, jnp.stack([att_src, att_dst]))
    ae2 = jnp.concatenate(
        [_ae(edge_attr, W_e, att_edge.reshape(D_H, 1)).reshape(NT, ET),
         jnp.full((NT, pad), -1e30, _f32)], axis=1).reshape(NT, NCH2, CH2)
    den_pad, out4 = _build_sc()(
        src2, dst2, ae2, a2[:, 0], a2[:, 1], h4)
    out = _bn(out4[:, :N, :], den_pad[:N].reshape(N, 1),
              gamma.reshape(2, 2 * DQ), beta.reshape(2, 2 * DQ))
    return (out, edge_index, edge_attr, batch)


# speedup vs baseline: 1.4360x; 1.4360x over previous
import functools

import jax
import jax.numpy as jnp
from jax import lax
from jax.experimental import pallas as pl
from jax.experimental.pallas import tpu as pltpu
from jax.experimental.pallas import tpu_sc as plsc

N = 10000
E = 160000
D_IN = 256
D_H = 256
D_E = 16

NT = 16
ET = E // NT
CH2 = 125
NCH2 = ET // CH2
NB = 3
NPAD = 10240
STRIP = NPAD // NT
DQ = 64
NR = 4
DBLK = NPAD // NR
DSUB = DBLK // NT

_f32 = jnp.float32
_i32 = jnp.int32


def _mm_body(x_ref, w_ref, att_ref, h_ref, a2_ref):
    xb = x_ref[...]
    hb = jnp.dot(xb, w_ref[...], preferred_element_type=_f32)
    for q in range(4):
        h_ref[q] = hb[:, q * DQ:(q + 1) * DQ]
    asb = jnp.sum(hb * att_ref[0][None, :], axis=1)
    adb = jnp.sum(hb * att_ref[1][None, :], axis=1)
    a2_ref[...] = jnp.stack([asb, adb], axis=1)


_MMBLK = 1000

_mm = pl.pallas_call(
    _mm_body,
    grid=(N // _MMBLK,),
    in_specs=[
        pl.BlockSpec((_MMBLK, D_IN), lambda i: (i, 0)),
        pl.BlockSpec((D_IN, D_H), lambda i: (0, 0)),
        pl.BlockSpec((2, D_H), lambda i: (0, 0)),
    ],
    out_specs=[
        pl.BlockSpec((4, _MMBLK, DQ), lambda i: (0, i, 0)),
        pl.BlockSpec((_MMBLK, 2), lambda i: (i, 0)),
    ],
    out_shape=[
        jax.ShapeDtypeStruct((4, N, DQ), _f32),
        jax.ShapeDtypeStruct((N, 2), _f32),
    ],
)


def _ae_body(ea_ref, we_ref, att_ref, out_ref):
    wv = jnp.dot(we_ref[...], att_ref[...], preferred_element_type=_f32)
    out_ref[...] = jnp.dot(ea_ref[...], wv, preferred_element_type=_f32)


_AEBLK = 8000

_ae = pl.pallas_call(
    _ae_body,
    grid=(E // _AEBLK,),
    in_specs=[
        pl.BlockSpec((_AEBLK, D_E), lambda i: (i, 0)),
        pl.BlockSpec((D_E, D_H), lambda i: (0, 0)),
        pl.BlockSpec((D_H, 1), lambda i: (0, 0)),
    ],
    out_specs=pl.BlockSpec((_AEBLK, 1), lambda i: (i, 0)),
    out_shape=jax.ShapeDtypeStruct((E, 1), _f32),
)


@functools.lru_cache(maxsize=1)
def _build_sc():
  mesh = plsc.VectorSubcoreMesh(core_axis_name="c", subcore_axis_name="s")

  @functools.partial(
    pl.kernel,
    mesh=mesh,
    compiler_params=pltpu.CompilerParams(
        needs_layout_passes=False, use_tc_tiling_on_sc=False),
    out_type=(
        jax.ShapeDtypeStruct((NPAD,), _f32),
        jax.ShapeDtypeStruct((4, NPAD, DQ), _f32),
    ),
    scratch_types=[
        pltpu.VMEM((N,), _f32),
        pltpu.VMEM((N,), _f32),
        pltpu.VMEM((NPAD,), _f32),
        pltpu.VMEM((DSUB,), _f32),
        pltpu.VMEM((DSUB,), _f32),
        pltpu.VMEM((NCH2, CH2), _i32),
        pltpu.VMEM((NCH2, CH2), _i32),
        pltpu.VMEM((NCH2, CH2), _f32),
        pltpu.VMEM((NB, CH2, DQ), _f32),
        pltpu.SemaphoreType.DMA((NB,)),
        pltpu.SemaphoreType.DMA((NB,)),
        pltpu.VMEM_SHARED((NT, DBLK), _f32),
        pltpu.VMEM_SHARED((NPAD, DQ), _f32),
    ],
  )
  def _sc(src2_h, dst2_h, ae2_h, as_h, ad_h, h2_h,
          den_o, out_o,
          asv, adv, denp, strip, tmp,
          srcm2, dstm2, aefm, rows4, semg, sems,
          stageb, acc):
    c = lax.axis_index("c")
    s = lax.axis_index("s")
    zeros16 = jnp.zeros((16,), _f32)
    iota16 = lax.iota(_i32, 16)

    cps = [pltpu.async_copy(src2_h.at[s], srcm2, semg.at[0]),
           pltpu.async_copy(dst2_h.at[s], dstm2, semg.at[1]),
           pltpu.async_copy(ae2_h.at[s], aefm, semg.at[2]),
           pltpu.async_copy(as_h, asv, sems.at[0]),
           pltpu.async_copy(ad_h, adv, sems.at[1])]
    for cp in cps:
        cp.wait()

    def _z(i, carry):
        denp[pl.ds(i * 16, 16)] = zeros16
        return carry
    lax.fori_loop(0, NPAD // 16, _z, 0)

    def _p1(i, carry):
        e = iota16 + i * 16
        er = e // CH2
        ec = e % CH2
        si = plsc.load_gather(srcm2, [er, ec])
        di = plsc.load_gather(dstm2, [er, ec])
        a = (plsc.load_gather(asv, [si])
             + plsc.load_gather(adv, [di])
             + plsc.load_gather(aefm, [er, ec]))
        a = jnp.maximum(a, a * 0.2)
        ex = jnp.exp(a)
        plsc.store_scatter(aefm, [er, ec], ex)
        plsc.addupdate_scatter(denp, [di], ex)
        return carry
    lax.fori_loop(0, ET // 16, _p1, 0)

    for kk in range(NR // 2):
        k = c * (NR // 2) + kk
        kbase = pl.multiple_of(k * DBLK, DBLK)
        pltpu.sync_copy(denp.at[pl.ds(kbase, DBLK)], stageb.at[s])
        plsc.subcore_barrier()
        sb = s * DSUB
        pltpu.sync_copy(stageb.at[0, pl.ds(sb, DSUB)], strip)

        def _red(j, carry):
            pltpu.sync_copy(stageb.at[j, pl.ds(sb, DSUB)], tmp)

            def _addv(i, carry2):
                sl = pl.ds(i * 16, 16)
                strip[sl] = strip[sl] + tmp[sl]
                return carry2
            lax.fori_loop(0, DSUB // 16, _addv, 0)
            return carry
        lax.fori_loop(1, NT, _red, 0)
        pltpu.sync_copy(strip, den_o.at[pl.ds(kbase + sb, DSUB)])
        plsc.subcore_barrier()

    def _pass(p, carry):
        qq = c * 2 + p
        h_q = h2_h.at[qq]
        out_q = out_o.at[qq]

        def _gather(g, b):
            return pltpu.async_copy(
                h_q.at[srcm2.at[g]], rows4.at[b], semg.at[b])

        def _gather_wait(g, b):
            pltpu.make_async_copy(
                h_q.at[srcm2.at[g]], rows4.at[b], semg.at[b]).wait()

        def _scatter(g, b):
            return pltpu.async_copy(
                rows4.at[b], acc.at[dstm2.at[g]], sems.at[b], add=True)

        def _scatter_wait(g, b):
            pltpu.make_async_copy(
                rows4.at[b], acc.at[dstm2.at[g]], sems.at[b]).wait()

        def _scale(g, b):
            gv = jnp.full((16,), g, _i32)

            @plsc.parallel_loop(0, CH2, unroll=5)
            def _row(r):
                sp = plsc.load_gather(aefm, [gv, jnp.full((16,), r, _i32)])
                for k in range(DQ // 16):
                    sl = pl.ds(k * 16, 16)
                    rows4[b, r, sl] = rows4[b, r, sl] * sp

        def _zr(r, carry2):
            for k in range(DQ // 16):
                rows4[0, r, pl.ds(k * 16, 16)] = zeros16
            return carry2
        lax.fori_loop(0, CH2, _zr, 0)
        zb = s * STRIP
        for k in range(STRIP // CH2):
            pltpu.sync_copy(rows4.at[0], acc.at[pl.ds(zb + k * CH2, CH2)])
        plsc.subcore_barrier()

        def _chunk(g, b):
            _gather_wait(g, b)
            _scale(g, b)
            _scatter(g, b)
            bp = (b + 2) % NB

            @pl.when(g >= NB - 2)
            def _svc_wait():
                _scatter_wait(g - (NB - 2), bp)

            @pl.when(g + 2 < NCH2)
            def _svc_gather():
                _gather(g + 2, bp)

        _gather(0, 0)
        _gather(1, 1)

        def _trip(q, carry2):
            for b in range(NB):
                _chunk(q * NB + b, b)
            return carry2
        lax.fori_loop(0, NCH2 // NB, _trip, 0)

        for g in range(NCH2 - NCH2 % NB, NCH2):
            _chunk(g, g % NB)
        for g in range(NCH2 - (NB - 2), NCH2):
            _scatter_wait(g, g % NB)

        plsc.subcore_barrier()
        ob = s * STRIP
        pltpu.sync_copy(acc.at[pl.ds(ob, STRIP)],
                        out_q.at[pl.ds(ob, STRIP)])
        plsc.subcore_barrier()
        return carry
    lax.fori_loop(0, 2, _pass, 0)

  return _sc


def _bn_body(acc_ref, den_ref, g_ref, b_ref, out_ref):
    j = pl.program_id(0)
    a = jnp.concatenate([acc_ref[0], acc_ref[1]], axis=1)
    d = den_ref[...] + 1e-16
    o = a / d
    mu = jnp.mean(o, axis=0, keepdims=True)
    var = jnp.mean((o - mu) ** 2, axis=0, keepdims=True)
    g = jnp.where(j == 0, g_ref[0:1, :], g_ref[1:2, :])
    b = jnp.where(j == 0, b_ref[0:1, :], b_ref[1:2, :])
    out_ref[...] = jnp.maximum(
        (o - mu) * lax.rsqrt(var + 1e-5) * g + b, 0.0)


_bn = pl.pallas_call(
    _bn_body,
    grid=(2,),
    in_specs=[
        pl.BlockSpec((2, N, DQ), lambda j: (j, 0, 0)),
        pl.BlockSpec((N, 1), lambda j: (0, 0)),
        pl.BlockSpec((2, 2 * DQ), lambda j: (0, 0)),
        pl.BlockSpec((2, 2 * DQ), lambda j: (0, 0)),
    ],
    out_specs=pl.BlockSpec((N, 2 * DQ), lambda j: (0, j)),
    out_shape=jax.ShapeDtypeStruct((N, D_H), _f32),
)


def kernel(x, edge_index, edge_attr, batch, W, att_src, att_dst, W_e,
           att_edge, bias, gamma, beta):
    del bias
    src2 = edge_index[0].reshape(NT, NCH2, CH2)
    dst2 = edge_index[1].reshape(NT, NCH2, CH2)
    h4, a2 = _mm(x, W, jnp.stack([att_src, att_dst]))
    ae2 = _ae(edge_attr, W_e, att_edge.reshape(D_H, 1)).reshape(
        NT, NCH2, CH2)
    den_pad, out4 = _build_sc()(
        src2, dst2, ae2, a2[:, 0], a2[:, 1], h4)
    out = _bn(out4[:, :N, :], den_pad[:N].reshape(N, 1),
              gamma.reshape(2, 2 * DQ), beta.reshape(2, 2 * DQ))
    return (out, edge_index, edge_attr, batch)
